# baseline (speedup 1.0000x reference)
import functools
import math

import jax
import jax.numpy as jnp
from jax import lax
from jax.experimental import pallas as pl
from jax.experimental.pallas import tpu as pltpu
from jax.experimental.pallas import tpu_sc as plsc

_L = 16


def _build_kernel(V, D, S, BBLK, NBH):
    scale = jnp.float32(math.sqrt(D))
    DH = D // 8
    SHI = S // 8
    mesh = plsc.VectorSubcoreMesh(core_axis_name="c", subcore_axis_name="s")

    @functools.partial(
        pl.kernel,
        mesh=mesh,
        out_type=jax.ShapeDtypeStruct((S, DH, NBH, 8, BBLK), jnp.float32),
        scratch_types=[
            pltpu.VMEM((SHI, 8, BBLK), jnp.int32),
            pltpu.VMEM((BBLK, D), jnp.float32),
            pltpu.VMEM((BBLK, D), jnp.float32),
            pltpu.VMEM((DH, 8, BBLK), jnp.float32),
            pltpu.VMEM((DH, 8, BBLK), jnp.float32),
            pltpu.SemaphoreType.DMA,
            pltpu.SemaphoreType.DMA,
            pltpu.SemaphoreType.DMA,
            pltpu.SemaphoreType.DMA,
        ],
        compiler_params=pltpu.CompilerParams(use_tc_tiling_on_sc=False,
                                             needs_layout_passes=False),
    )
    def k(t4_hbm, table_hbm, out_hbm, idx_v, rows0, rows1, tb0, tb1,
          g0, g1, s0, s1):
        w = lax.axis_index("s") * 2 + lax.axis_index("c")
        pltpu.sync_copy(t4_hbm.at[:, w], idx_v)

        iota = lax.iota(jnp.int32, _L)
        row_ids = [iota + jnp.int32(blk * _L) for blk in range(BBLK // _L)]

        def idx_slice(step):
            return idx_v.at[lax.shift_right_logical(step, 1 + 1 + 1),
                            lax.bitwise_and(step, jnp.int32(7))]

        def start_gather(step, rows_b, gsem):
            pltpu.async_copy(table_hbm.at[idx_slice(step)], rows_b, gsem)

        def wait_gather(step, rows_b, gsem):
            pltpu.make_async_copy(table_hbm.at[idx_slice(step)], rows_b,
                                  gsem).wait()

        def transpose_scale(rows_b, tb_b):
            def dh_body(dh, c):
                for dl in range(8):
                    col = jnp.broadcast_to(dh * 8 + dl, (_L,)).astype(jnp.int32)
                    for blk in range(BBLK // _L):
                        v = plsc.load_gather(rows_b, [row_ids[blk], col])
                        tb_b[dh, dl, pl.ds(blk * _L, _L)] = v * scale
                return c
            lax.fori_loop(0, DH, dh_body, 0)

        def start_stores(step, tb_b, ssem):
            for dh in range(DH):
                pltpu.async_copy(tb_b.at[dh], out_hbm.at[step, dh, w], ssem)

        def wait_stores(step, tb_b, ssem):
            for dh in range(DH):
                pltpu.make_async_copy(tb_b.at[dh], out_hbm.at[step, dh, w],
                                      ssem).wait()

        start_gather(jnp.int32(0), rows0, g0)
        start_gather(jnp.int32(1), rows1, g1)

        def outer(kk, carry):
            for b, (rows_b, tb_b, gsem, ssem) in enumerate(
                    [(rows0, tb0, g0, s0), (rows1, tb1, g1, s1)]):
                step = kk * 2 + b
                wait_gather(step, rows_b, gsem)

                @pl.when(kk > 0)
                def _():
                    wait_stores(step, tb_b, ssem)

                transpose_scale(rows_b, tb_b)

                @pl.when(step + 2 < S)
                def _():
                    start_gather(step + 2, rows_b, gsem)

                start_stores(step, tb_b, ssem)
            return carry

        lax.fori_loop(0, S // 2, outer, 0)
        wait_stores(jnp.int32(S - 2), tb0, s0)
        wait_stores(jnp.int32(S - 1), tb1, s1)

    return k


def kernel(tokens, table):
    B0, S = tokens.shape
    V, D = table.shape
    NBH = B0 // 128
    t4 = tokens.T.reshape(S // 8, 8, NBH, 128).transpose(0, 2, 1, 3)
    out5 = _build_kernel(V, D, S, 128, NBH)(t4, table)
    return out5.transpose(2, 4, 0, 1, 3).reshape(B0, S, D)


# speedup vs baseline: 1.6595x; 1.6595x over previous
import functools
import math

import jax
import jax.numpy as jnp
from jax import lax
from jax.experimental import pallas as pl
from jax.experimental.pallas import tpu as pltpu
from jax.experimental.pallas import tpu_sc as plsc

_L = 16


def _build_kernel(V, D, S, BBLK, NBH):
    scale = jnp.float32(math.sqrt(D))
    DH = D // 8
    SHI = S // 8
    mesh = plsc.VectorSubcoreMesh(core_axis_name="c", subcore_axis_name="s")

    @functools.partial(
        pl.kernel,
        mesh=mesh,
        out_type=jax.ShapeDtypeStruct((S, DH, NBH, 8, BBLK), jnp.float32),
        scratch_types=[
            pltpu.VMEM((SHI, 8, BBLK), jnp.int32),
            pltpu.VMEM((BBLK, 2 * D), jnp.float32),
            pltpu.VMEM((BBLK, 2 * D), jnp.float32),
            pltpu.VMEM((D, BBLK), jnp.float32),
            pltpu.VMEM((D, BBLK), jnp.float32),
            pltpu.VMEM((BBLK,), jnp.int32),
            pltpu.VMEM((BBLK,), jnp.int32),
            pltpu.SemaphoreType.DMA,
            pltpu.SemaphoreType.DMA,
            pltpu.SemaphoreType.DMA,
            pltpu.SemaphoreType.DMA,
        ],
        compiler_params=pltpu.CompilerParams(use_tc_tiling_on_sc=False,
                                             needs_layout_passes=False),
    )
    def k(t4_hbm, tab2_hbm, out_hbm, idx_v, rows0, rows1, tb0, tb1,
          i20, i21, g0, g1, s0, s1):
        w = lax.axis_index("s") * 2 + lax.axis_index("c")
        pltpu.sync_copy(t4_hbm.at[:, w], idx_v)

        iota = lax.iota(jnp.int32, _L)
        row_ids = [iota + jnp.int32(blk * _L) for blk in range(BBLK // _L)]
        diag = [lax.bitwise_and(iota + jnp.int32(kd), jnp.int32(_L - 1))
                for kd in range(_L)]

        def raw_vec(step, blk):
            return idx_v[lax.shift_right_logical(step, 3),
                         lax.bitwise_and(step, jnp.int32(7)),
                         pl.ds(blk * _L, _L)]

        def fill_idx2(step, i2_b):
            for blk in range(BBLK // _L):
                i2_b[pl.ds(blk * _L, _L)] = lax.shift_right_logical(
                    raw_vec(step, blk), 1)

        def gather_copy(rows_b, i2_b, gsem):
            return pltpu.make_async_copy(tab2_hbm.at[i2_b], rows_b, gsem)

        def transpose_scale(step, rows_b, tb_b):
            par64 = [lax.bitwise_and(raw_vec(step, blk), jnp.int32(1)) * D
                     for blk in range(BBLK // _L)]

            def d0_body(d0g, c):
                d0 = d0g * _L
                for kd in range(_L):
                    fvec = diag[kd] + d0
                    for blk in range(BBLK // _L):
                        col = fvec + par64[blk]
                        v = plsc.load_gather(rows_b, [row_ids[blk], col])
                        plsc.store_scatter(tb_b, [fvec, row_ids[blk]],
                                           v * scale)
                return c
            lax.fori_loop(0, D // _L, d0_body, 0)

        def start_stores(step, tb_b, ssem):
            for dh in range(DH):
                pltpu.async_copy(tb_b.at[pl.ds(dh * 8, 8)],
                                 out_hbm.at[step, dh, w], ssem)

        def wait_stores(step, tb_b, ssem):
            for dh in range(DH):
                pltpu.make_async_copy(tb_b.at[pl.ds(dh * 8, 8)],
                                     out_hbm.at[step, dh, w], ssem).wait()

        fill_idx2(jnp.int32(0), i20)
        gather_copy(rows0, i20, g0).start()
        fill_idx2(jnp.int32(1), i21)
        gather_copy(rows1, i21, g1).start()

        def outer(kk, carry):
            for b, (rows_b, tb_b, i2_b, gsem, ssem) in enumerate(
                    [(rows0, tb0, i20, g0, s0), (rows1, tb1, i21, g1, s1)]):
                step = kk * 2 + b
                gather_copy(rows_b, i2_b, gsem).wait()

                @pl.when(kk > 0)
                def _():
                    wait_stores(step, tb_b, ssem)

                transpose_scale(step, rows_b, tb_b)

                @pl.when(step + 2 < S)
                def _():
                    fill_idx2(step + 2, i2_b)
                    gather_copy(rows_b, i2_b, gsem).start()

                start_stores(step, tb_b, ssem)
            return carry

        lax.fori_loop(0, S // 2, outer, 0)
        wait_stores(jnp.int32(S - 2), tb0, s0)
        wait_stores(jnp.int32(S - 1), tb1, s1)

    return k


def kernel(tokens, table):
    B0, S = tokens.shape
    V, D = table.shape
    NBH = B0 // 128
    t4 = tokens.T.reshape(S // 8, 8, NBH, 128).transpose(0, 2, 1, 3)
    tab2 = table.reshape(V // 2, 2 * D)
    out5 = _build_kernel(V, D, S, 128, NBH)(t4, tab2)
    return out5.transpose(2, 4, 0, 1, 3).reshape(B0, S, D)
